# baseline (speedup 1.0000x reference)
import functools

import jax
import jax.numpy as jnp
from jax import lax
from jax.experimental import pallas as pl
from jax.experimental.pallas import tpu as pltpu
from jax.experimental.pallas import tpu_sc as plsc

VOCAB = 1000000
DIM = 32
B = 4096
L = 200

_R = 5
_W = 128


def _make_kernel():
    info = plsc.get_sparse_core_info()
    nc = info.num_cores
    nw = nc * info.num_subcores
    assert nw * _W == B and L % _R == 0

    mesh = plsc.VectorSubcoreMesh(core_axis_name="c", subcore_axis_name="s")

    @functools.partial(
        pl.kernel,
        mesh=mesh,
        out_type=jax.ShapeDtypeStruct((L * DIM, B), jnp.float32),
        scratch_types=(
            [pltpu.VMEM((L, _W), jnp.int32)]
            + [pltpu.VMEM((_W, DIM), jnp.float32) for _ in range(_R)]
            + [pltpu.VMEM((DIM, _W), jnp.float32) for _ in range(_R)]
            + [pltpu.SemaphoreType.DMA] * (2 * _R + 1)
        ),
        compiler_params=pltpu.CompilerParams(use_tc_tiling_on_sc=False,
                                             needs_layout_passes=False),
    )
    def k(xt_hbm, tbl_hbm, out_hbm, *refs):
        idxs = refs[0]
        rows = refs[1:1 + _R]
        touts = refs[1 + _R:1 + 2 * _R]
        sem_g = refs[1 + 2 * _R:1 + 3 * _R]
        sem_s = refs[1 + 3 * _R:1 + 4 * _R]
        sem_i = refs[1 + 4 * _R]

        wid = lax.axis_index("s") * nc + lax.axis_index("c")
        col0 = wid * _W

        pltpu.async_copy(xt_hbm.at[:, pl.ds(col0, _W)], idxs, sem_i).wait()

        iota = lax.iota(jnp.int32, 16)

        def g_start(l, s):
            pltpu.async_copy(tbl_hbm.at[idxs.at[l]], rows[s], sem_g[s])

        def g_wait(l, s):
            pltpu.make_async_copy(tbl_hbm.at[idxs.at[l]], rows[s],
                                  sem_g[s]).wait()

        def out_slice(l):
            return out_hbm.at[pl.ds(l * DIM, DIM), pl.ds(col0, _W)]

        def s_start(l, s):
            pltpu.async_copy(touts[s], out_slice(l), sem_s[s])

        def s_wait(l, s):
            pltpu.make_async_copy(touts[s], out_slice(l), sem_s[s]).wait()

        def transpose(s):
            def per_d(d, carry):
                dcol = jnp.full((16,), 0, jnp.int32) + d
                for g in range(_W // 16):
                    v = plsc.load_gather(rows[s], [g * 16 + iota, dcol])
                    touts[s][d, pl.ds(g * 16, 16)] = v
                return carry

            lax.fori_loop(0, DIM, per_d, 0, unroll=False)

        def step(l, s, wait_store, start_gather):
            g_wait(l, s)
            if start_gather:
                g_start(l + (_R - 1), (s + _R - 1) % _R)
            if wait_store:
                s_wait(l - _R, s)
            transpose(s)
            s_start(l, s)

        for s in range(_R - 1):
            g_start(s, s)
        for i in range(_R):
            step(i, i, False, True)

        def block(blk, carry):
            for i in range(_R):
                step(blk * _R + i, i, True, True)
            return carry

        lax.fori_loop(1, L // _R - 1, block, 0)

        for i in range(_R):
            l = L - _R + i
            step(l, i, True, i == 0)
        for i in range(_R):
            s_wait(L - _R + i, i)

    return k


_gather = _make_kernel()


@jax.jit
def kernel(x, weight):
    xt = jnp.swapaxes(x.astype(jnp.int32), 0, 1)
    out = _gather(xt, weight)
    return jnp.transpose(out.reshape(L, DIM, B), (2, 0, 1))


# speedup vs baseline: 1.9438x; 1.9438x over previous
import functools

import jax
import jax.numpy as jnp
from jax import lax
from jax.experimental import pallas as pl
from jax.experimental.pallas import tpu as pltpu
from jax.experimental.pallas import tpu_sc as plsc

VOCAB = 1000000
DIM = 32
B = 4096
L = 200

_R = 5
_W = 128
_TP = 131


def _make_kernel():
    info = plsc.get_sparse_core_info()
    nc = info.num_cores
    nw = nc * info.num_subcores
    assert nw * _W == B and L % _R == 0

    mesh = plsc.VectorSubcoreMesh(core_axis_name="c", subcore_axis_name="s")

    @functools.partial(
        pl.kernel,
        mesh=mesh,
        out_type=jax.ShapeDtypeStruct((L, DIM // 8, B // _W, 8, _W),
                                      jnp.float32),
        scratch_types=(
            [pltpu.VMEM((L // 8, 8, _W), jnp.int32)]
            + [pltpu.VMEM((_W, DIM), jnp.float32) for _ in range(_R)]
            + [pltpu.VMEM((DIM // 8, 8, _TP), jnp.float32)
               for _ in range(_R)]
            + [pltpu.SemaphoreType.DMA] * (2 * _R + 1)
        ),
        compiler_params=pltpu.CompilerParams(use_tc_tiling_on_sc=False,
                                             needs_layout_passes=False),
    )
    def k(xq_hbm, tbl_hbm, out_hbm, *refs):
        idxs = refs[0]
        rows = refs[1:1 + _R]
        touts = refs[1 + _R:1 + 2 * _R]
        sem_g = refs[1 + 2 * _R:1 + 3 * _R]
        sem_s = refs[1 + 3 * _R:1 + 4 * _R]
        sem_i = refs[1 + 4 * _R]

        wid = lax.axis_index("s") * nc + lax.axis_index("c")

        pltpu.async_copy(xq_hbm.at[:, wid], idxs, sem_i).wait()

        iota = lax.iota(jnp.int32, 16)

        def idx_ref(l):
            return idxs.at[l // 8, l % 8]

        def g_start(l, s):
            pltpu.async_copy(tbl_hbm.at[idx_ref(l)], rows[s], sem_g[s])

        def g_wait(l, s):
            pltpu.make_async_copy(tbl_hbm.at[idx_ref(l)], rows[s],
                                  sem_g[s]).wait()

        def out_slice(l):
            return out_hbm.at[l, :, wid]

        def tout_src(s):
            return touts[s].at[:, :, pl.ds(0, _W)]

        def s_start(l, s):
            pltpu.async_copy(tout_src(s), out_slice(l), sem_s[s])

        def s_wait(l, s):
            pltpu.make_async_copy(tout_src(s), out_slice(l), sem_s[s]).wait()

        d_hi1, d_lo1 = iota // 8, iota % 8
        d_hi2, d_lo2 = (16 + iota) // 8, (16 + iota) % 8

        def transpose(s):
            def per_j(j, carry):
                jcol = jnp.full((16,), 0, jnp.int32) + j
                v1 = rows[s][j, pl.ds(0, 16)]
                v2 = rows[s][j, pl.ds(16, 16)]
                plsc.store_scatter(touts[s], [d_hi1, d_lo1, jcol], v1)
                plsc.store_scatter(touts[s], [d_hi2, d_lo2, jcol], v2)
                return carry

            lax.fori_loop(0, _W, per_j, 0, unroll=4)

        def step(l, s, wait_store, start_gather):
            g_wait(l, s)
            if start_gather:
                g_start(l + (_R - 1), (s + _R - 1) % _R)
            if wait_store:
                s_wait(l - _R, s)
            transpose(s)
            s_start(l, s)

        for s in range(_R - 1):
            g_start(s, s)
        for i in range(_R):
            step(i, i, False, True)

        def block(blk, carry):
            for i in range(_R):
                step(blk * _R + i, i, True, True)
            return carry

        lax.fori_loop(1, L // _R - 1, block, 0)

        for i in range(_R):
            l = L - _R + i
            step(l, i, True, i == 0)
        for i in range(_R):
            s_wait(L - _R + i, i)

    return k


_gather = _make_kernel()


@jax.jit
def kernel(x, weight):
    xq = (jnp.swapaxes(x.astype(jnp.int32), 0, 1)
          .reshape(L // 8, 8, B // _W, _W)
          .transpose(0, 2, 1, 3))
    o5 = _gather(xq, weight)
    return (o5.transpose(0, 1, 3, 2, 4)
            .reshape(L, DIM, B)
            .transpose(2, 0, 1))
